# Initial kernel scaffold; baseline (speedup 1.0000x reference)
import jax
import jax.numpy as jnp
from jax.experimental import pallas as pl


def kernel(X, edge_index, edge_weight, weight_n, weight_e, query_w, key_w, W_att, b_att, W_out, b_out):
    raise NotImplementedError("write your pallas kernel here")


# speedup vs baseline: 2.3572x; 2.3572x over previous
import functools

import jax
import jax.numpy as jnp
from jax import lax
from jax.experimental import pallas as pl
from jax.experimental.pallas import tpu as pltpu
from jax.experimental.pallas import tpu_sc as plsc

N_NODES = 10000
N_EDGES = 320000
D = 128

ROW_BLK = 400
N_TC_BLOCKS = N_NODES // ROW_BLK

NC = 2
NS = 16
N_WORKERS = NC * NS
EDGES_PER_TILE = N_EDGES // N_WORKERS
CHUNK = 16
SUP = 400
N_SUP = EDGES_PER_TILE // SUP
N_CHUNKS = SUP // CHUNK
STRIPE = 624
TAIL = N_NODES - NS * STRIPE
ZROWS = 48


def _node_stage(x_in, wn, qw, kw, watt, we_ref, batt_ref, x_out, aq_out,
                ak_out, cvec_out):
    x = jnp.dot(x_in[...], wn[...], preferred_element_type=jnp.float32)
    x_out[...] = x
    qa = jnp.dot(qw[...], watt[0:D, :], preferred_element_type=jnp.float32)
    ka = jnp.dot(kw[...], watt[D:2 * D, :], preferred_element_type=jnp.float32)
    aq_out[...] = jnp.dot(x, qa, preferred_element_type=jnp.float32) + batt_ref[...]
    ak_out[...] = jnp.dot(x, ka, preferred_element_type=jnp.float32)
    cv = jnp.dot(we_ref[...], watt[2 * D:3 * D, :],
                 preferred_element_type=jnp.float32)
    cvec_out[...] = jnp.broadcast_to(cv, (8, D))


def _update_stage(x_ref, a0_ref, a1_ref, wo1, wo2, bo, out_ref):
    x = x_ref[...]
    a = a0_ref[...] + a1_ref[...]
    out_ref[...] = (x + jnp.dot(x, wo1[...], preferred_element_type=jnp.float32)
                    + jnp.dot(a, wo2[...], preferred_element_type=jnp.float32)
                    + bo[...])


def _edge_stage(x_hbm, aq_hbm, ak_hbm, src_hbm, dst_hbm, ew_hbm, params_hbm,
                out_hbm, aq_tab, ak_tab, params_v, src_all, dst_all, ew_all,
                rows_v, zbuf, aggr_sh, sem):
    c = lax.axis_index("c")
    s = lax.axis_index("s")
    wid = c * NS + s
    base0 = pl.multiple_of(wid * EDGES_PER_TILE, 8)

    pltpu.sync_copy(aq_hbm, aq_tab)
    pltpu.sync_copy(ak_hbm, ak_tab)
    pltpu.sync_copy(params_hbm, params_v)

    def _zero_row(i, carry):
        for d in range(8):
            zbuf[i, pl.ds(d * 16, 16)] = jnp.zeros((16,), jnp.float32)
        return carry
    lax.fori_loop(0, ZROWS, _zero_row, 0)
    row0 = pl.multiple_of(s * STRIPE, 8)
    for k in range(STRIPE // ZROWS):
        pltpu.sync_copy(
            zbuf, aggr_sh.at[pl.ds(pl.multiple_of(row0 + k * ZROWS, 8), ZROWS)])

    @pl.when(s == 0)
    def _zero_tail():
        pltpu.sync_copy(zbuf.at[pl.ds(0, TAIL)],
                        aggr_sh.at[pl.ds(NS * STRIPE, TAIL)])
    plsc.subcore_barrier()

    we = [params_v[pl.ds(d * 16, 16)] for d in range(8)]
    c_const = params_v[pl.ds(D, 16)][0]

    def _super(sp, carry):
        base = pl.multiple_of(base0 + sp * SUP, 8)
        pltpu.sync_copy(src_hbm.at[pl.ds(base, SUP)], src_all)
        pltpu.sync_copy(dst_hbm.at[pl.ds(base, SUP)], dst_all)
        pltpu.sync_copy(ew_hbm.at[pl.ds(base, SUP)], ew_all)

        def _chunk(g, carry2):
            sl = pl.ds(g * CHUNK, CHUNK)
            src16 = src_all[sl]
            dst16 = dst_all[sl]
            ew16 = ew_all[sl]
            pltpu.async_copy(x_hbm.at[src16], rows_v, sem).wait()

            aq16 = plsc.load_gather(aq_tab, [src16])
            ak16 = plsc.load_gather(ak_tab, [dst16])
            z = aq16 + ak16 + c_const * ew16
            att16 = 1.0 / (1.0 + jnp.exp(-z))

            for j in range(CHUNK):
                att_e = att16[j]
                ew_e = ew16[j]
                for d in range(8):
                    dsl = pl.ds(d * 16, 16)
                    gz = jnp.exp(-(ew_e * we[d]))
                    coef = att_e / (1.0 + gz)
                    rows_v[j, dsl] = rows_v[j, dsl] * coef

            pltpu.sync_copy(rows_v, aggr_sh.at[dst16], add=True)
            return carry2
        lax.fori_loop(0, N_CHUNKS, _chunk, 0)
        return carry
    lax.fori_loop(0, N_SUP, _super, 0)

    plsc.subcore_barrier()
    for k in range(STRIPE // ZROWS):
        row = pl.multiple_of(row0 + k * ZROWS, 8)
        pltpu.sync_copy(aggr_sh.at[pl.ds(row, ZROWS)], zbuf)
        pltpu.sync_copy(zbuf, out_hbm.at[c, pl.ds(row, ZROWS)])

    @pl.when(s == 0)
    def _copy_tail():
        pltpu.sync_copy(aggr_sh.at[pl.ds(NS * STRIPE, TAIL)],
                        zbuf.at[pl.ds(0, TAIL)])
        pltpu.sync_copy(zbuf.at[pl.ds(0, TAIL)],
                        out_hbm.at[c, pl.ds(NS * STRIPE, TAIL)])


def _run_edge_stage(x, aq, ak, src, dst, ew, params):
    mesh = plsc.VectorSubcoreMesh(core_axis_name="c", subcore_axis_name="s")
    f = pl.kernel(
        _edge_stage,
        out_type=jax.ShapeDtypeStruct((NC, N_NODES, D), jnp.float32),
        mesh=mesh,
        scratch_types=[
            pltpu.VMEM((N_NODES,), jnp.float32),
            pltpu.VMEM((N_NODES,), jnp.float32),
            pltpu.VMEM((144,), jnp.float32),
            pltpu.VMEM((SUP,), jnp.int32),
            pltpu.VMEM((SUP,), jnp.int32),
            pltpu.VMEM((SUP,), jnp.float32),
            pltpu.VMEM((CHUNK, D), jnp.float32),
            pltpu.VMEM((ZROWS, D), jnp.float32),
            pltpu.VMEM_SHARED((N_NODES, D), jnp.float32),
            pltpu.SemaphoreType.DMA,
        ],
        compiler_params=pltpu.CompilerParams(needs_layout_passes=False),
    )
    return f(x, aq, ak, src, dst, ew, params)


def kernel(X, edge_index, edge_weight, weight_n, weight_e, query_w, key_w,
           W_att, b_att, W_out, b_out):
    src = edge_index[0].astype(jnp.int32)
    dst = edge_index[1].astype(jnp.int32)
    ew = edge_weight.astype(jnp.float32)

    full = lambda shape: pl.BlockSpec(shape, lambda i: (0, 0))
    node = pl.pallas_call(
        _node_stage,
        grid=(N_TC_BLOCKS,),
        in_specs=[
            pl.BlockSpec((ROW_BLK, D), lambda i: (i, 0)),
            full((D, D)), full((D, D)), full((D, D)), full((3 * D, 1)),
            full((1, D)), full((1, 1)),
        ],
        out_specs=[
            pl.BlockSpec((ROW_BLK, D), lambda i: (i, 0)),
            pl.BlockSpec((ROW_BLK, 1), lambda i: (i, 0)),
            pl.BlockSpec((ROW_BLK, 1), lambda i: (i, 0)),
            pl.BlockSpec((8, D), lambda i: (0, 0)),
        ],
        out_shape=[
            jax.ShapeDtypeStruct((N_NODES, D), jnp.float32),
            jax.ShapeDtypeStruct((N_NODES, 1), jnp.float32),
            jax.ShapeDtypeStruct((N_NODES, 1), jnp.float32),
            jax.ShapeDtypeStruct((8, D), jnp.float32),
        ],
    )
    x, aq, ak, cvec = node(X, weight_n, query_w, key_w, W_att, weight_e,
                           b_att.reshape(1, 1))

    params = jnp.concatenate([weight_e[0], cvec[0, 0:1],
                              jnp.zeros((15,), jnp.float32)])
    aggr2 = _run_edge_stage(x, aq.reshape(N_NODES), ak.reshape(N_NODES),
                            src, dst, ew, params)

    upd = pl.pallas_call(
        _update_stage,
        grid=(N_TC_BLOCKS,),
        in_specs=[
            pl.BlockSpec((ROW_BLK, D), lambda i: (i, 0)),
            pl.BlockSpec((ROW_BLK, D), lambda i: (i, 0)),
            pl.BlockSpec((ROW_BLK, D), lambda i: (i, 0)),
            full((D, D)), full((D, D)), full((1, D)),
        ],
        out_specs=pl.BlockSpec((ROW_BLK, D), lambda i: (i, 0)),
        out_shape=jax.ShapeDtypeStruct((N_NODES, D), jnp.float32),
    )
    return upd(x, aggr2[0], aggr2[1], W_out[:D], W_out[D:], b_out.reshape(1, D))
